# baseline (speedup 1.0000x reference)
import jax
import jax.numpy as jnp
from jax import lax
from jax.experimental import pallas as pl
from jax.experimental.pallas import tpu as pltpu
from jax.experimental.pallas import tpu_sc as plsc

VOCAB = 100000
EMB = 128
BATCH = 4096
SEQ = 200

_NC = 2
_NS = 16
_NW = _NC * _NS
_TOTAL = BATCH * SEQ
_PER_W = _TOTAL // _NW
_GRP = 128
_NGRP = _PER_W // _GRP
_NBUF = 3
_SLAB = 2 * _GRP
_NSTEP = _PER_W // _SLAB


def _emb_kernel(x_hbm, table_hbm, out_hbm, idx_v, *rest):
    bufs = rest[:_NBUF]
    gsems = rest[_NBUF:2 * _NBUF]
    osems = rest[2 * _NBUF:3 * _NBUF]
    wid = lax.axis_index("s") * _NC + lax.axis_index("c")
    base = wid * _PER_W

    pltpu.sync_copy(x_hbm.at[wid], idx_v)

    def gathers(s, b, start):
        for h in range(2):
            cp = pltpu.make_async_copy(
                table_hbm.at[idx_v.at[2 * s + h]],
                bufs[b].at[pl.ds(h * _GRP, _GRP)], gsems[b])
            cp.start() if start else cp.wait()

    def out(s, b, start):
        cp = pltpu.make_async_copy(
            bufs[b], out_hbm.at[pl.ds(base + s * _SLAB, _SLAB)], osems[b])
        cp.start() if start else cp.wait()

    for b in range(_NBUF):
        gathers(b, b, True)

    def body(p, carry):
        s0 = p * _NBUF
        for b in range(_NBUF):
            gathers(s0 + b, b, False)
            out(s0 + b, b, True)
        for b in range(_NBUF):
            s_next = s0 + _NBUF + b

            @pl.when(s_next < _NSTEP)
            def _(b=b, s_next=s_next):
                out(s_next - _NBUF, b, False)
                gathers(s_next, b, True)
        return carry

    lax.fori_loop(0, _NSTEP // _NBUF, body, 0)
    gathers(_NSTEP - 1, 0, False)
    out(_NSTEP - 1, 0, True)
    out(_NSTEP - 3, 1, False)
    out(_NSTEP - 2, 2, False)
    out(_NSTEP - 1, 0, False)


def _mask_kernel(x_ref, o_ref):
    o_ref[...] = jnp.where(x_ref[...] != 0,
                           jnp.float32(1.0), jnp.float32(0.0))


@jax.jit
def kernel(x, table):
    x32 = x.astype(jnp.int32)
    xw = x32.reshape(_NW, _NGRP, _GRP)
    mesh = plsc.VectorSubcoreMesh(core_axis_name="c", subcore_axis_name="s")
    out = pl.kernel(
        _emb_kernel,
        mesh=mesh,
        out_type=jax.ShapeDtypeStruct((_TOTAL, EMB), jnp.float32),
        scratch_types=[
            pltpu.VMEM((_NGRP, _GRP), jnp.int32),
            *[pltpu.VMEM((_SLAB, EMB), jnp.float32) for _ in range(_NBUF)],
            *[pltpu.SemaphoreType.DMA for _ in range(2 * _NBUF)],
        ],
    )(xw, table)
    mask = pl.pallas_call(
        _mask_kernel,
        out_shape=jax.ShapeDtypeStruct((BATCH, SEQ), jnp.float32),
        grid=(8,),
        in_specs=[pl.BlockSpec((BATCH // 8, SEQ), lambda i: (i, 0))],
        out_specs=pl.BlockSpec((BATCH // 8, SEQ), lambda i: (i, 0)),
    )(x32)
    return out.reshape(BATCH, SEQ, EMB), mask
